# baseline (speedup 1.0000x reference)
import jax
import jax.numpy as jnp
from jax.experimental import pallas as pl
from jax.experimental.pallas import tpu as pltpu

_EPS = 1e-10
_SMOOTH = 1e-5
_W_CE = 1.0
_W_DICE = 1.0
_LN2 = 0.6931471805599453

_Q_INTER = 0
_Q_PREDO = 16
_Q_GROUND = 32
_Q_CE = 48
_NQ = 49
_ROWS = _NQ * 8


def _dice_ce_body(t_ref, p_ref, out_ref):
    j = pl.program_id(1)

    @pl.when(j == 0)
    def _():
        out_ref[...] = jnp.zeros_like(out_ref)

    n_classes = p_ref.shape[1]
    bh = p_ref.shape[2]

    def rs(x):
        return x[0:8] + x[8:16]

    qn = 8
    qs = 16

    zero = jnp.zeros((8, 128), jnp.float32)
    ce_acc = zero
    for c in range(n_classes):
        a_i0 = a_i1 = a_p0 = a_p1 = a_g0 = a_g1 = zero
        for h in range(bh):
            for q in range(qn):
                sl = slice(qs * q, qs * q + qs)
                t0 = t_ref[0, 0, h, sl, :]
                t1 = t_ref[1, 0, h, sl, :]
                m0 = t0 == c
                m1 = t1 == c
                m0f = m0.astype(jnp.float32)
                m1f = m1.astype(jnp.float32)
                p0 = p_ref[0, c, h, sl, :]
                p1 = p_ref[1, c, h, sl, :]
                lq = jnp.log(p0 * p1 + _EPS * _EPS)
                a_i0 = a_i0 + rs(jnp.where(m0, p0, 0.0))
                a_i1 = a_i1 + rs(jnp.where(m1, p1, 0.0))
                a_p0 = a_p0 + rs(p0)
                a_p1 = a_p1 + rs(p1)
                a_g0 = a_g0 + rs(m0f)
                a_g1 = a_g1 + rs(m1f)
                ce_acc = ce_acc + rs((m0f + m1f) * lq)
        out_ref[0, 8 * (_Q_INTER + c):8 * (_Q_INTER + c) + 8, :] += a_i0
        out_ref[0, 8 * (_Q_INTER + 8 + c):8 * (_Q_INTER + 8 + c) + 8, :] += a_i1
        out_ref[0, 8 * (_Q_PREDO + c):8 * (_Q_PREDO + c) + 8, :] += a_p0
        out_ref[0, 8 * (_Q_PREDO + 8 + c):8 * (_Q_PREDO + 8 + c) + 8, :] += a_p1
        out_ref[0, 8 * (_Q_GROUND + c):8 * (_Q_GROUND + c) + 8, :] += a_g0
        out_ref[0, 8 * (_Q_GROUND + 8 + c):8 * (_Q_GROUND + 8 + c) + 8, :] += a_g1
    out_ref[0, 8 * _Q_CE:8 * _Q_CE + 8, :] += ce_acc


def kernel(pred, target):
    B, N, H, W, D = pred.shape
    BH = 16
    NCORE = 2
    J = H // (NCORE * BH)

    out = pl.pallas_call(
        _dice_ce_body,
        out_shape=jax.ShapeDtypeStruct((NCORE, _ROWS, 128), jnp.float32),
        grid=(NCORE, J),
        in_specs=[
            pl.BlockSpec((B, 1, BH, W, D), lambda i, j: (0, 0, i * J + j, 0, 0)),
            pl.BlockSpec((B, N, BH, W, D), lambda i, j: (0, 0, i * J + j, 0, 0)),
        ],
        out_specs=pl.BlockSpec((1, _ROWS, 128), lambda i, j: (i, 0, 0)),
        compiler_params=pltpu.CompilerParams(
            dimension_semantics=("parallel", "arbitrary"),
            vmem_limit_bytes=56 * 1024 * 1024,
        ),
        name="dice_ce_fused",
    )(target, pred)

    vals = jnp.sum(out, axis=(0, 2)).reshape(_NQ, 8).sum(axis=1)
    inter = vals[_Q_INTER:_Q_INTER + 16].reshape(2, 8)
    pred_o = vals[_Q_PREDO:_Q_PREDO + 16].reshape(2, 8)
    ground_o = vals[_Q_GROUND:_Q_GROUND + 16].reshape(2, 8)
    ce_sum = vals[_Q_CE]

    hwd = H * W * D
    celoss = -ce_sum / (B * B * hwd)
    dice = jnp.mean(1.0 - (2.0 * inter + _SMOOTH) / (ground_o + pred_o + _SMOOTH))
    return _W_CE * celoss + _W_DICE * dice


# speedup vs baseline: 1.0079x; 1.0079x over previous
import jax
import jax.numpy as jnp
from jax.experimental import pallas as pl
from jax.experimental.pallas import tpu as pltpu

_EPS = 1e-10
_SMOOTH = 1e-5
_W_CE = 1.0
_W_DICE = 1.0
_LN2 = 0.6931471805599453

_Q_INTER = 0
_Q_PREDO = 16
_Q_GROUND = 32
_Q_CE = 48
_NQ = 49
_ROWS = _NQ * 8


def _dice_ce_body(t_ref, p_ref, out_ref):
    j = pl.program_id(1)

    @pl.when(j == 0)
    def _():
        out_ref[...] = jnp.zeros_like(out_ref)

    n_classes = p_ref.shape[1]
    bh = p_ref.shape[2]

    def rs(x):
        return (x[0:8] + x[8:16]) + (x[16:24] + x[24:32])

    qn = 4
    qs = 32

    zero = jnp.zeros((8, 128), jnp.float32)
    ce_acc = zero
    for c in range(n_classes):
        a_i0 = a_i1 = a_p0 = a_p1 = a_g0 = a_g1 = zero
        for h in range(bh):
            for q in range(qn):
                sl = slice(qs * q, qs * q + qs)
                t0 = t_ref[0, 0, h, sl, :]
                t1 = t_ref[1, 0, h, sl, :]
                m0 = t0 == c
                m1 = t1 == c
                m0f = m0.astype(jnp.float32)
                m1f = m1.astype(jnp.float32)
                p0 = p_ref[0, c, h, sl, :]
                p1 = p_ref[1, c, h, sl, :]
                lq = jnp.log(p0 * p1 + _EPS * _EPS)
                a_i0 = a_i0 + rs(jnp.where(m0, p0, 0.0))
                a_i1 = a_i1 + rs(jnp.where(m1, p1, 0.0))
                a_p0 = a_p0 + rs(p0)
                a_p1 = a_p1 + rs(p1)
                a_g0 = a_g0 + rs(m0f)
                a_g1 = a_g1 + rs(m1f)
                ce_acc = ce_acc + rs((m0f + m1f) * lq)
        out_ref[0, 8 * (_Q_INTER + c):8 * (_Q_INTER + c) + 8, :] += a_i0
        out_ref[0, 8 * (_Q_INTER + 8 + c):8 * (_Q_INTER + 8 + c) + 8, :] += a_i1
        out_ref[0, 8 * (_Q_PREDO + c):8 * (_Q_PREDO + c) + 8, :] += a_p0
        out_ref[0, 8 * (_Q_PREDO + 8 + c):8 * (_Q_PREDO + 8 + c) + 8, :] += a_p1
        out_ref[0, 8 * (_Q_GROUND + c):8 * (_Q_GROUND + c) + 8, :] += a_g0
        out_ref[0, 8 * (_Q_GROUND + 8 + c):8 * (_Q_GROUND + 8 + c) + 8, :] += a_g1
    out_ref[0, 8 * _Q_CE:8 * _Q_CE + 8, :] += ce_acc


def kernel(pred, target):
    B, N, H, W, D = pred.shape
    BH = 16
    NCORE = 2
    J = H // (NCORE * BH)

    out = pl.pallas_call(
        _dice_ce_body,
        out_shape=jax.ShapeDtypeStruct((NCORE, _ROWS, 128), jnp.float32),
        grid=(NCORE, J),
        in_specs=[
            pl.BlockSpec((B, 1, BH, W, D), lambda i, j: (0, 0, i * J + j, 0, 0)),
            pl.BlockSpec((B, N, BH, W, D), lambda i, j: (0, 0, i * J + j, 0, 0)),
        ],
        out_specs=pl.BlockSpec((1, _ROWS, 128), lambda i, j: (i, 0, 0)),
        compiler_params=pltpu.CompilerParams(
            dimension_semantics=("parallel", "arbitrary"),
            vmem_limit_bytes=56 * 1024 * 1024,
        ),
        name="dice_ce_fused",
    )(target, pred)

    vals = jnp.sum(out, axis=(0, 2)).reshape(_NQ, 8).sum(axis=1)
    inter = vals[_Q_INTER:_Q_INTER + 16].reshape(2, 8)
    pred_o = vals[_Q_PREDO:_Q_PREDO + 16].reshape(2, 8)
    ground_o = vals[_Q_GROUND:_Q_GROUND + 16].reshape(2, 8)
    ce_sum = vals[_Q_CE]

    hwd = H * W * D
    celoss = -ce_sum / (B * B * hwd)
    dice = jnp.mean(1.0 - (2.0 * inter + _SMOOTH) / (ground_o + pred_o + _SMOOTH))
    return _W_CE * celoss + _W_DICE * dice
